# baseline (speedup 1.0000x reference)
import functools

import jax
import jax.numpy as jnp
from jax import lax
from jax.experimental import pallas as pl
from jax.experimental.pallas import tpu as pltpu
from jax.experimental.pallas import tpu_sc as plsc

_B, _L = 16384, 50
_WEATHER, _SEASON, _TDIM, _EDIM = 10, 4, 8, 64
_P = _B * _L
_NSEG = _TDIM + 1
_NJ = _WEATHER * _SEASON * _NSEG
_NW2 = _EDIM // 2

_NC, _NS = 2, 16
_NWRK = _NC * _NS
_PPW = _P // _NWRK
_CH = 1280
_NCHUNK = _PPW // _CH


def _sc_body(wid_hbm, sid_hbm, t_hbm, j_hbm, a_hbm, th_hbm, out_hbm,
             w0, w1, s0, s1, t0, t1, o0, o1, j_v, a_v, th_v,
             si0, si1, so0, so1):
    wkr = lax.axis_index("s") * _NC + lax.axis_index("c")
    base0 = wkr * _PPW
    ins = ((w0, s0, t0, si0), (w1, s1, t1, si1))
    outs = ((o0, so0), (o1, so1))

    def start_in(ci, b):
        pos0 = base0 + ci * _CH
        pltpu.async_copy(wid_hbm.at[pl.ds(pos0, _CH)], ins[b][0], ins[b][3])
        pltpu.async_copy(sid_hbm.at[pl.ds(pos0, _CH)], ins[b][1], ins[b][3])
        pltpu.async_copy(t_hbm.at[pl.ds(pos0, _CH)], ins[b][2], ins[b][3])

    def wait_in(b):
        for r in ins[b][0:3]:
            pltpu.make_async_copy(wid_hbm.at[pl.ds(base0, _CH)], r,
                                  ins[b][3]).wait()

    def wait_out(b):
        pltpu.make_async_copy(outs[b][0],
                              out_hbm.at[pl.ds(base0 * _NW2, _CH * _NW2)],
                              outs[b][1]).wait()

    start_in(0, 0)
    start_in(1, 1)
    pltpu.sync_copy(j_hbm, j_v)
    pltpu.sync_copy(a_hbm, a_v)
    pltpu.sync_copy(th_hbm, th_v)

    lane32 = lax.iota(jnp.int32, 16) * _NW2
    th_vecs = [th_v[pl.ds(i * 16, 16)] for i in range(_TDIM)]
    himask = jnp.int32(-65536)

    def compute(b):
        widb, sidb, tb, _ = ins[b]
        outb = outs[b][0]

        @plsc.parallel_loop(0, _CH // 16, unroll=2)
        def group(g):
            gb = g * 16
            wv = widb[pl.ds(gb, 16)]
            sv = sidb[pl.ds(gb, 16)]
            tv = tb[pl.ds(gb, 16)]
            seg = jnp.where(tv > th_vecs[0], 1, 0)
            for i in range(1, _TDIM):
                seg = seg + jnp.where(tv > th_vecs[i], 1, 0)
            jb = (wv * (_SEASON * _NSEG * _NW2) + sv * (_NSEG * _NW2)
                  + seg * _NW2)
            ab = seg * _NW2
            ob = lane32 + gb * _NW2
            for d0 in range(0, _NW2, 8):
                jw = [plsc.load_gather(j_v, [jb + (d0 + k)]) for k in range(8)]
                aw = [plsc.load_gather(a_v, [ab + (d0 + k)]) for k in range(8)]
                wo = []
                for k in range(8):
                    jlo = plsc.bitcast(jw[k] << 16, jnp.float32)
                    jhi = plsc.bitcast(jw[k] & himask, jnp.float32)
                    alo = plsc.bitcast(aw[k] << 16, jnp.float32)
                    ahi = plsc.bitcast(aw[k] & himask, jnp.float32)
                    vlo = jlo + tv * alo
                    vhi = jhi + tv * ahi
                    pk = plsc.pack(vlo, vhi,
                                   format=plsc.PackFormat.INTERLEAVED)
                    wo.append(plsc.bitcast(pk, jnp.int32))
                for k in range(8):
                    plsc.store_scatter(outb, [ob + (d0 + k)], wo[k])

    def outer(oi, carry):
        for b in range(2):
            ci = oi * 2 + b
            pos0 = base0 + ci * _CH

            @pl.when(ci >= 2)
            def _():
                wait_out(b)

            wait_in(b)
            compute(b)
            pltpu.async_copy(outs[b][0],
                             out_hbm.at[pl.ds(pos0 * _NW2, _CH * _NW2)],
                             outs[b][1])

            @pl.when(ci + 2 < _NCHUNK)
            def _():
                start_in(ci + 2, b)
        return carry

    lax.fori_loop(0, _NCHUNK // 2, outer, 0)
    wait_out(0)
    wait_out(1)


_mesh = plsc.VectorSubcoreMesh(core_axis_name="c", subcore_axis_name="s")
_sc_call = pl.kernel(
    _sc_body,
    out_type=jax.ShapeDtypeStruct((_P * _NW2,), jnp.int32),
    mesh=_mesh,
    compiler_params=pltpu.CompilerParams(needs_layout_passes=False),
    scratch_types=[
        pltpu.VMEM((_CH,), jnp.int32), pltpu.VMEM((_CH,), jnp.int32),
        pltpu.VMEM((_CH,), jnp.int32), pltpu.VMEM((_CH,), jnp.int32),
        pltpu.VMEM((_CH,), jnp.float32), pltpu.VMEM((_CH,), jnp.float32),
        pltpu.VMEM((_CH * _NW2,), jnp.int32),
        pltpu.VMEM((_CH * _NW2,), jnp.int32),
        pltpu.VMEM((_NJ * _NW2,), jnp.int32),
        pltpu.VMEM((_NSEG * _NW2,), jnp.int32),
        pltpu.VMEM((_TDIM * 16,), jnp.float32),
        pltpu.SemaphoreType.DMA, pltpu.SemaphoreType.DMA,
        pltpu.SemaphoreType.DMA, pltpu.SemaphoreType.DMA,
    ],
)


def _fold_tables(W_weather, W_season, W_t1, b_t1, W_t2, b_t2, W_c, b_c):
    f32 = jnp.float32
    Wc = W_c.astype(f32)
    Tw = W_weather.astype(f32) @ Wc[0:16]
    Ts = W_season.astype(f32) @ Wc[16:32]
    W2c = W_t2.astype(f32) @ Wc[32:64]
    btot = b_t2.astype(f32) @ Wc[32:64] + b_c.astype(f32)
    w1 = W_t1.astype(f32)[0]
    b1 = b_t1.astype(f32)
    safe_w1 = jnp.where(w1 != 0, w1, 1.0)
    theta = jnp.where(w1 != 0, -b1 / safe_w1, jnp.inf)
    order = jnp.argsort(theta)
    theta_s = theta[order]
    rank = jnp.argsort(order)
    k = jnp.arange(_NSEG)[:, None]
    active = jnp.where(w1[None, :] > 0, rank[None, :] < k,
                       jnp.where(w1[None, :] < 0, rank[None, :] >= k,
                                 b1[None, :] > 0))
    act = active.astype(f32)
    A = (act * w1[None, :]) @ W2c
    Bs = (act * b1[None, :]) @ W2c + btot
    Tws = (Tw[:, None, :] + Ts[None, :, :]).reshape(_WEATHER * _SEASON, _EDIM)
    J = (Tws[:, None, :] + Bs[None, :, :]).reshape(_NJ, _EDIM)
    return J, A, theta_s


def _pack_pairs(M):
    w16 = jax.lax.bitcast_convert_type(M.astype(jnp.bfloat16), jnp.uint16)
    w32 = w16.astype(jnp.uint32)
    packed = w32[:, 0::2] | (w32[:, 1::2] << 16)
    return jax.lax.bitcast_convert_type(packed, jnp.int32).reshape(-1)


def kernel(weather_ids, time_of_day, season_ids, W_weather, W_season,
           W_t1, b_t1, W_t2, b_t2, W_c, b_c):
    J, A, theta_s = _fold_tables(W_weather, W_season, W_t1, b_t1,
                                 W_t2, b_t2, W_c, b_c)
    wid = weather_ids.reshape(_P).astype(jnp.int32)
    sid = season_ids.reshape(_P).astype(jnp.int32)
    t = time_of_day.reshape(_P).astype(jnp.float32)
    jf = _pack_pairs(J)
    af = _pack_pairs(A)
    thb = jnp.broadcast_to(theta_s[:, None], (_TDIM, 16)).reshape(_TDIM * 16)
    out_w = _sc_call(wid, sid, t, jf, af, thb)
    out_bf = jax.lax.bitcast_convert_type(out_w.reshape(_P, _NW2),
                                          jnp.bfloat16)
    return out_bf.astype(jnp.float32).reshape(_B, _L, _EDIM)


# speedup vs baseline: 1.5277x; 1.5277x over previous
import functools

import jax
import jax.numpy as jnp
from jax import lax
from jax.experimental import pallas as pl
from jax.experimental.pallas import tpu as pltpu
from jax.experimental.pallas import tpu_sc as plsc

_B, _L = 16384, 50
_WEATHER, _SEASON, _TDIM, _EDIM = 10, 4, 8, 64
_P = _B * _L
_NSEG = _TDIM + 1
_NJ = _WEATHER * _SEASON * _NSEG
_NW2 = _EDIM // 2

_NC, _NS = 2, 16
_NWRK = _NC * _NS
_PPW = _P // _NWRK
_CH = 1280
_NCHUNK = _PPW // _CH


def _sc_body(wid_hbm, sid_hbm, t_hbm, j_hbm, a_hbm, th_hbm, out_hbm,
             w0, w1, s0, s1, t0, t1, o0, o1, j_v, a_v, th_v,
             si0, si1, so0, so1):
    wkr = lax.axis_index("s") * _NC + lax.axis_index("c")
    base0 = wkr * _PPW
    ins = ((w0, s0, t0, si0), (w1, s1, t1, si1))
    outs = ((o0, so0), (o1, so1))

    def start_in(ci, b):
        pos0 = base0 + ci * _CH
        pltpu.async_copy(wid_hbm.at[pl.ds(pos0, _CH)], ins[b][0], ins[b][3])
        pltpu.async_copy(sid_hbm.at[pl.ds(pos0, _CH)], ins[b][1], ins[b][3])
        pltpu.async_copy(t_hbm.at[pl.ds(pos0, _CH)], ins[b][2], ins[b][3])

    def wait_in(b):
        for r in ins[b][0:3]:
            pltpu.make_async_copy(wid_hbm.at[pl.ds(base0, _CH)], r,
                                  ins[b][3]).wait()

    def wait_out(b):
        pltpu.make_async_copy(outs[b][0],
                              out_hbm.at[pl.ds(base0 * _NW2, _CH * _NW2)],
                              outs[b][1]).wait()

    start_in(0, 0)
    start_in(1, 1)
    pltpu.sync_copy(j_hbm, j_v)
    pltpu.sync_copy(a_hbm, a_v)
    pltpu.sync_copy(th_hbm, th_v)

    lane32 = lax.iota(jnp.int32, 16) * _NW2
    th_vecs = [th_v[pl.ds(i * 16, 16)] for i in range(_TDIM)]
    himask = jnp.int32(-65536)

    def compute(b):
        widb, sidb, tb, _ = ins[b]
        outb = outs[b][0]

        @plsc.parallel_loop(0, _CH // 16, unroll=2)
        def group(g):
            gb = g * 16
            wv = widb[pl.ds(gb, 16)]
            sv = sidb[pl.ds(gb, 16)]
            tv = tb[pl.ds(gb, 16)]
            seg = jnp.where(tv > th_vecs[0], 1, 0)
            for i in range(1, _TDIM):
                seg = seg + jnp.where(tv > th_vecs[i], 1, 0)
            jb = (wv * (_SEASON * _NSEG * _NW2) + sv * (_NSEG * _NW2)
                  + seg * _NW2)
            ab = seg * _NW2
            ob = lane32 + gb * _NW2
            for d0 in range(0, _NW2, 8):
                jw = [plsc.load_gather(j_v, [jb + (d0 + k)]) for k in range(8)]
                aw = [plsc.load_gather(a_v, [ab + (d0 + k)]) for k in range(8)]
                wo = []
                for k in range(8):
                    jlo = plsc.bitcast(jw[k] << 16, jnp.float32)
                    jhi = plsc.bitcast(jw[k] & himask, jnp.float32)
                    alo = plsc.bitcast(aw[k] << 16, jnp.float32)
                    ahi = plsc.bitcast(aw[k] & himask, jnp.float32)
                    vlo = jlo + tv * alo
                    vhi = jhi + tv * ahi
                    pk = plsc.pack(vlo, vhi,
                                   format=plsc.PackFormat.INTERLEAVED)
                    wo.append(plsc.bitcast(pk, jnp.int32))
                for k in range(8):
                    plsc.store_scatter(outb, [ob + (d0 + k)], wo[k])

    def outer(oi, carry):
        for b in range(2):
            ci = oi * 2 + b
            pos0 = base0 + ci * _CH

            @pl.when(ci >= 2)
            def _():
                wait_out(b)

            wait_in(b)
            compute(b)
            pltpu.async_copy(outs[b][0],
                             out_hbm.at[pl.ds(pos0 * _NW2, _CH * _NW2)],
                             outs[b][1])

            @pl.when(ci + 2 < _NCHUNK)
            def _():
                start_in(ci + 2, b)
        return carry

    lax.fori_loop(0, _NCHUNK // 2, outer, 0)
    wait_out(0)
    wait_out(1)


_mesh = plsc.VectorSubcoreMesh(core_axis_name="c", subcore_axis_name="s")
_sc_call = pl.kernel(
    _sc_body,
    out_type=jax.ShapeDtypeStruct((_P * _NW2,), jnp.int32),
    mesh=_mesh,
    compiler_params=pltpu.CompilerParams(needs_layout_passes=False),
    scratch_types=[
        pltpu.VMEM((_CH,), jnp.int32), pltpu.VMEM((_CH,), jnp.int32),
        pltpu.VMEM((_CH,), jnp.int32), pltpu.VMEM((_CH,), jnp.int32),
        pltpu.VMEM((_CH,), jnp.float32), pltpu.VMEM((_CH,), jnp.float32),
        pltpu.VMEM((_CH * _NW2,), jnp.int32),
        pltpu.VMEM((_CH * _NW2,), jnp.int32),
        pltpu.VMEM((_NJ * _NW2,), jnp.int32),
        pltpu.VMEM((_NSEG * _NW2,), jnp.int32),
        pltpu.VMEM((_TDIM * 16,), jnp.float32),
        pltpu.SemaphoreType.DMA, pltpu.SemaphoreType.DMA,
        pltpu.SemaphoreType.DMA, pltpu.SemaphoreType.DMA,
    ],
)


def _fold_tables(W_weather, W_season, W_t1, b_t1, W_t2, b_t2, W_c, b_c):
    f32 = jnp.float32
    Wc = W_c.astype(f32)
    Tw = W_weather.astype(f32) @ Wc[0:16]
    Ts = W_season.astype(f32) @ Wc[16:32]
    W2c = W_t2.astype(f32) @ Wc[32:64]
    btot = b_t2.astype(f32) @ Wc[32:64] + b_c.astype(f32)
    w1 = W_t1.astype(f32)[0]
    b1 = b_t1.astype(f32)
    safe_w1 = jnp.where(w1 != 0, w1, 1.0)
    theta = jnp.where(w1 != 0, -b1 / safe_w1, jnp.inf)
    order = jnp.argsort(theta)
    theta_s = theta[order]
    rank = jnp.argsort(order)
    k = jnp.arange(_NSEG)[:, None]
    active = jnp.where(w1[None, :] > 0, rank[None, :] < k,
                       jnp.where(w1[None, :] < 0, rank[None, :] >= k,
                                 b1[None, :] > 0))
    act = active.astype(f32)
    A = (act * w1[None, :]) @ W2c
    Bs = (act * b1[None, :]) @ W2c + btot
    Tws = (Tw[:, None, :] + Ts[None, :, :]).reshape(_WEATHER * _SEASON, _EDIM)
    J = (Tws[:, None, :] + Bs[None, :, :]).reshape(_NJ, _EDIM)
    return J, A, theta_s


def _pack_pairs(M):
    w16 = jax.lax.bitcast_convert_type(M.astype(jnp.bfloat16), jnp.uint16)
    w32 = w16.astype(jnp.uint32)
    packed = w32[:, :_NW2] | (w32[:, _NW2:] << 16)
    return jax.lax.bitcast_convert_type(packed, jnp.int32).reshape(-1)


def kernel(weather_ids, time_of_day, season_ids, W_weather, W_season,
           W_t1, b_t1, W_t2, b_t2, W_c, b_c):
    J, A, theta_s = _fold_tables(W_weather, W_season, W_t1, b_t1,
                                 W_t2, b_t2, W_c, b_c)
    wid = weather_ids.reshape(_P).astype(jnp.int32)
    sid = season_ids.reshape(_P).astype(jnp.int32)
    t = time_of_day.reshape(_P).astype(jnp.float32)
    jf = _pack_pairs(J)
    af = _pack_pairs(A)
    thb = jnp.broadcast_to(theta_s[:, None], (_TDIM, 16)).reshape(_TDIM * 16)
    out_w = _sc_call(wid, sid, t, jf, af, thb).reshape(_P, _NW2)
    lo = jax.lax.bitcast_convert_type(jnp.left_shift(out_w, 16),
                                      jnp.float32)
    hi = jax.lax.bitcast_convert_type(out_w & jnp.int32(-65536),
                                      jnp.float32)
    return jnp.concatenate([lo, hi], axis=-1).reshape(_B, _L, _EDIM)


# speedup vs baseline: 1.5283x; 1.0004x over previous
import functools

import jax
import jax.numpy as jnp
from jax import lax
from jax.experimental import pallas as pl
from jax.experimental.pallas import tpu as pltpu
from jax.experimental.pallas import tpu_sc as plsc

_B, _L = 16384, 50
_WEATHER, _SEASON, _TDIM, _EDIM = 10, 4, 8, 64
_P = _B * _L
_NSEG = _TDIM + 1
_NJ = _WEATHER * _SEASON * _NSEG
_NW2 = _EDIM // 2

_NC, _NS = 2, 16
_NWRK = _NC * _NS
_PPW = _P // _NWRK
_CH = 1280
_NCHUNK = _PPW // _CH


def _sc_body(wid_hbm, sid_hbm, t_hbm, j_hbm, a_hbm, th_hbm, out_hbm,
             w0, w1, s0, s1, t0, t1, o0, o1, j_v, a_v, th_v,
             si0, si1, so0, so1):
    wkr = lax.axis_index("s") * _NC + lax.axis_index("c")
    base0 = wkr * _PPW
    ins = ((w0, s0, t0, si0), (w1, s1, t1, si1))
    outs = ((o0, so0), (o1, so1))

    def start_in(ci, b):
        pos0 = base0 + ci * _CH
        pltpu.async_copy(wid_hbm.at[pl.ds(pos0, _CH)], ins[b][0], ins[b][3])
        pltpu.async_copy(sid_hbm.at[pl.ds(pos0, _CH)], ins[b][1], ins[b][3])
        pltpu.async_copy(t_hbm.at[pl.ds(pos0, _CH)], ins[b][2], ins[b][3])

    def wait_in(b):
        for r in ins[b][0:3]:
            pltpu.make_async_copy(wid_hbm.at[pl.ds(base0, _CH)], r,
                                  ins[b][3]).wait()

    def wait_out(b):
        pltpu.make_async_copy(outs[b][0],
                              out_hbm.at[pl.ds(base0 * _NW2, _CH * _NW2)],
                              outs[b][1]).wait()

    start_in(0, 0)
    start_in(1, 1)
    pltpu.sync_copy(j_hbm, j_v)
    pltpu.sync_copy(a_hbm, a_v)
    pltpu.sync_copy(th_hbm, th_v)

    lane32 = lax.iota(jnp.int32, 16) * _NW2
    th_vecs = [th_v[pl.ds(i * 16, 16)] for i in range(_TDIM)]
    himask = jnp.int32(-65536)

    def compute(b):
        widb, sidb, tb, _ = ins[b]
        outb = outs[b][0]

        @plsc.parallel_loop(0, _CH // 16, unroll=2)
        def group(g):
            gb = g * 16
            wv = widb[pl.ds(gb, 16)]
            sv = sidb[pl.ds(gb, 16)]
            tv = tb[pl.ds(gb, 16)]
            seg = jnp.where(tv > th_vecs[0], 1, 0)
            for i in range(1, _TDIM):
                seg = seg + jnp.where(tv > th_vecs[i], 1, 0)
            jb = (wv * (_SEASON * _NSEG * _NW2) + sv * (_NSEG * _NW2)
                  + seg * _NW2)
            ab = seg * _NW2
            ob = lane32 + gb * _NW2
            for d0 in range(0, _NW2, 8):
                jw = [plsc.load_gather(j_v, [jb + (d0 + k)]) for k in range(8)]
                aw = [plsc.load_gather(a_v, [ab + (d0 + k)]) for k in range(8)]
                wo = []
                for k in range(8):
                    jlo = plsc.bitcast(jw[k] << 16, jnp.float32)
                    jhi = plsc.bitcast(jw[k] & himask, jnp.float32)
                    alo = plsc.bitcast(aw[k] << 16, jnp.float32)
                    ahi = plsc.bitcast(aw[k] & himask, jnp.float32)
                    vlo = jlo + tv * alo
                    vhi = jhi + tv * ahi
                    pk = plsc.pack(vlo, vhi,
                                   format=plsc.PackFormat.INTERLEAVED)
                    wo.append(plsc.bitcast(pk, jnp.int32))
                for k in range(8):
                    plsc.store_scatter(outb, [ob + (d0 + k)], wo[k])

    def outer(oi, carry):
        for b in range(2):
            ci = oi * 2 + b
            pos0 = base0 + ci * _CH

            @pl.when(ci >= 2)
            def _():
                wait_out(b)

            wait_in(b)
            compute(b)
            pltpu.async_copy(outs[b][0],
                             out_hbm.at[pl.ds(pos0 * _NW2, _CH * _NW2)],
                             outs[b][1])

            @pl.when(ci + 2 < _NCHUNK)
            def _():
                start_in(ci + 2, b)
        return carry

    lax.fori_loop(0, _NCHUNK // 2, outer, 0)
    wait_out(0)
    wait_out(1)


_mesh = plsc.VectorSubcoreMesh(core_axis_name="c", subcore_axis_name="s")
_sc_call = pl.kernel(
    _sc_body,
    out_type=jax.ShapeDtypeStruct((_P * _NW2,), jnp.int32),
    mesh=_mesh,
    compiler_params=pltpu.CompilerParams(needs_layout_passes=False),
    scratch_types=[
        pltpu.VMEM((_CH,), jnp.int32), pltpu.VMEM((_CH,), jnp.int32),
        pltpu.VMEM((_CH,), jnp.int32), pltpu.VMEM((_CH,), jnp.int32),
        pltpu.VMEM((_CH,), jnp.float32), pltpu.VMEM((_CH,), jnp.float32),
        pltpu.VMEM((_CH * _NW2,), jnp.int32),
        pltpu.VMEM((_CH * _NW2,), jnp.int32),
        pltpu.VMEM((_NJ * _NW2,), jnp.int32),
        pltpu.VMEM((_NSEG * _NW2,), jnp.int32),
        pltpu.VMEM((_TDIM * 16,), jnp.float32),
        pltpu.SemaphoreType.DMA, pltpu.SemaphoreType.DMA,
        pltpu.SemaphoreType.DMA, pltpu.SemaphoreType.DMA,
    ],
)


def _fold_tables(W_weather, W_season, W_t1, b_t1, W_t2, b_t2, W_c, b_c):
    f32 = jnp.float32
    Wc = W_c.astype(f32)
    Tw = W_weather.astype(f32) @ Wc[0:16]
    Ts = W_season.astype(f32) @ Wc[16:32]
    W2c = W_t2.astype(f32) @ Wc[32:64]
    btot = b_t2.astype(f32) @ Wc[32:64] + b_c.astype(f32)
    w1 = W_t1.astype(f32)[0]
    b1 = b_t1.astype(f32)
    safe_w1 = jnp.where(w1 != 0, w1, 1.0)
    theta = jnp.where(w1 != 0, -b1 / safe_w1, jnp.inf)
    order = jnp.argsort(theta)
    theta_s = theta[order]
    rank = jnp.argsort(order)
    k = jnp.arange(_NSEG)[:, None]
    active = jnp.where(w1[None, :] > 0, rank[None, :] < k,
                       jnp.where(w1[None, :] < 0, rank[None, :] >= k,
                                 b1[None, :] > 0))
    act = active.astype(f32)
    A = (act * w1[None, :]) @ W2c
    Bs = (act * b1[None, :]) @ W2c + btot
    Tws = (Tw[:, None, :] + Ts[None, :, :]).reshape(_WEATHER * _SEASON, _EDIM)
    J = (Tws[:, None, :] + Bs[None, :, :]).reshape(_NJ, _EDIM)
    return J, A, theta_s


def _pack_pairs(M):
    w16 = jax.lax.bitcast_convert_type(M.astype(jnp.bfloat16), jnp.uint16)
    w32 = w16.astype(jnp.uint32)
    packed = w32[:, :_NW2] | (w32[:, _NW2:] << 16)
    return jax.lax.bitcast_convert_type(packed, jnp.int32).reshape(-1)


def kernel(weather_ids, time_of_day, season_ids, W_weather, W_season,
           W_t1, b_t1, W_t2, b_t2, W_c, b_c):
    J, A, theta_s = _fold_tables(W_weather, W_season, W_t1, b_t1,
                                 W_t2, b_t2, W_c, b_c)
    wid = weather_ids.reshape(_P).astype(jnp.int32)
    sid = season_ids.reshape(_P).astype(jnp.int32)
    t = time_of_day.reshape(_P).astype(jnp.float32)
    jf = _pack_pairs(J)
    af = _pack_pairs(A)
    thb = jnp.broadcast_to(theta_s[:, None], (_TDIM, 16)).reshape(_TDIM * 16)
    out_w = _sc_call(wid, sid, t, jf, af, thb).reshape(_P // 4, 128)
    lo = jax.lax.bitcast_convert_type(jnp.left_shift(out_w, 16),
                                      jnp.float32).reshape(_P, _NW2)
    hi = jax.lax.bitcast_convert_type(out_w & jnp.int32(-65536),
                                      jnp.float32).reshape(_P, _NW2)
    return jnp.concatenate([lo, hi], axis=-1).reshape(_B, _L, _EDIM)


# speedup vs baseline: 1.5298x; 1.0010x over previous
import jax
import jax.numpy as jnp
from jax import lax
from jax.experimental import pallas as pl
from jax.experimental.pallas import tpu as pltpu
from jax.experimental.pallas import tpu_sc as plsc

_B, _L = 16384, 50
_WEATHER, _SEASON, _TDIM, _EDIM = 10, 4, 8, 64
_P = _B * _L
_NSEG = _TDIM + 1
_NJ = _WEATHER * _SEASON * _NSEG
_NW2 = _EDIM // 2

_NC, _NS = 2, 16
_NWRK = _NC * _NS
_PPW = _P // _NWRK
_CH = 1280
_NCHUNK = _PPW // _CH


def _sc_body(wid_hbm, sid_hbm, t_hbm, j_hbm, a_hbm, th_hbm, out_hbm,
             w0, w1, s0, s1, t0, t1, o0, o1, j_v, a_v, th_v,
             si0, si1, so0, so1):
    wkr = lax.axis_index("s") * _NC + lax.axis_index("c")
    base0 = wkr * _PPW
    ins = ((w0, s0, t0, si0), (w1, s1, t1, si1))
    outs = ((o0, so0), (o1, so1))

    def start_in(ci, b):
        pos0 = base0 + ci * _CH
        pltpu.async_copy(wid_hbm.at[pl.ds(pos0, _CH)], ins[b][0], ins[b][3])
        pltpu.async_copy(sid_hbm.at[pl.ds(pos0, _CH)], ins[b][1], ins[b][3])
        pltpu.async_copy(t_hbm.at[pl.ds(pos0, _CH)], ins[b][2], ins[b][3])

    def wait_in(b):
        for r in ins[b][0:3]:
            pltpu.make_async_copy(wid_hbm.at[pl.ds(base0, _CH)], r,
                                  ins[b][3]).wait()

    def wait_out(b):
        pltpu.make_async_copy(outs[b][0],
                              out_hbm.at[pl.ds(base0 * _NW2, _CH * _NW2)],
                              outs[b][1]).wait()

    start_in(0, 0)
    start_in(1, 1)
    pltpu.sync_copy(j_hbm, j_v)
    pltpu.sync_copy(a_hbm, a_v)
    pltpu.sync_copy(th_hbm, th_v)

    lane32 = lax.iota(jnp.int32, 16) * _NW2
    th_vecs = [th_v[pl.ds(i * 16, 16)] for i in range(_TDIM)]
    himask = jnp.int32(-65536)

    def compute(b):
        widb, sidb, tb, _ = ins[b]
        outb = outs[b][0]

        @plsc.parallel_loop(0, _CH // 16, unroll=2)
        def group(g):
            gb = g * 16
            wv = widb[pl.ds(gb, 16)]
            sv = sidb[pl.ds(gb, 16)]
            tv = tb[pl.ds(gb, 16)]
            seg = jnp.where(tv > th_vecs[0], 1, 0)
            for i in range(1, _TDIM):
                seg = seg + jnp.where(tv > th_vecs[i], 1, 0)
            jb = (wv * (_SEASON * _NSEG * _NW2) + sv * (_NSEG * _NW2)
                  + seg * _NW2)
            ab = seg * _NW2
            ob = lane32 + gb * _NW2
            for d0 in range(0, _NW2, 8):
                jw = [plsc.load_gather(j_v, [jb + (d0 + k)]) for k in range(8)]
                aw = [plsc.load_gather(a_v, [ab + (d0 + k)]) for k in range(8)]
                wo = []
                for k in range(8):
                    jlo = plsc.bitcast(jw[k] << 16, jnp.float32)
                    jhi = plsc.bitcast(jw[k] & himask, jnp.float32)
                    alo = plsc.bitcast(aw[k] << 16, jnp.float32)
                    ahi = plsc.bitcast(aw[k] & himask, jnp.float32)
                    vlo = jlo + tv * alo
                    vhi = jhi + tv * ahi
                    pk = plsc.pack(vlo, vhi,
                                   format=plsc.PackFormat.INTERLEAVED)
                    wo.append(plsc.bitcast(pk, jnp.int32))
                for k in range(8):
                    plsc.store_scatter(outb, [ob + (d0 + k)], wo[k])

    def outer(oi, carry):
        for b in range(2):
            ci = oi * 2 + b
            pos0 = base0 + ci * _CH

            @pl.when(ci >= 2)
            def _():
                wait_out(b)

            wait_in(b)
            compute(b)
            pltpu.async_copy(outs[b][0],
                             out_hbm.at[pl.ds(pos0 * _NW2, _CH * _NW2)],
                             outs[b][1])

            @pl.when(ci + 2 < _NCHUNK)
            def _():
                start_in(ci + 2, b)
        return carry

    lax.fori_loop(0, _NCHUNK // 2, outer, 0)
    wait_out(0)
    wait_out(1)


_mesh = plsc.VectorSubcoreMesh(core_axis_name="c", subcore_axis_name="s")
_sc_call = pl.kernel(
    _sc_body,
    out_type=jax.ShapeDtypeStruct((_P * _NW2,), jnp.int32),
    mesh=_mesh,
    compiler_params=pltpu.CompilerParams(needs_layout_passes=False),
    scratch_types=[
        pltpu.VMEM((_CH,), jnp.int32), pltpu.VMEM((_CH,), jnp.int32),
        pltpu.VMEM((_CH,), jnp.int32), pltpu.VMEM((_CH,), jnp.int32),
        pltpu.VMEM((_CH,), jnp.float32), pltpu.VMEM((_CH,), jnp.float32),
        pltpu.VMEM((_CH * _NW2,), jnp.int32),
        pltpu.VMEM((_CH * _NW2,), jnp.int32),
        pltpu.VMEM((_NJ * _NW2,), jnp.int32),
        pltpu.VMEM((_NSEG * _NW2,), jnp.int32),
        pltpu.VMEM((_TDIM * 16,), jnp.float32),
        pltpu.SemaphoreType.DMA, pltpu.SemaphoreType.DMA,
        pltpu.SemaphoreType.DMA, pltpu.SemaphoreType.DMA,
    ],
)


def _fold_tables(W_weather, W_season, W_t1, b_t1, W_t2, b_t2, W_c, b_c):
    f32 = jnp.float32
    Wc = W_c.astype(f32)
    Tw = W_weather.astype(f32) @ Wc[0:16]
    Ts = W_season.astype(f32) @ Wc[16:32]
    W2c = W_t2.astype(f32) @ Wc[32:64]
    btot = b_t2.astype(f32) @ Wc[32:64] + b_c.astype(f32)
    w1 = W_t1.astype(f32)[0]
    b1 = b_t1.astype(f32)
    safe_w1 = jnp.where(w1 != 0, w1, 1.0)
    theta = jnp.where(w1 != 0, -b1 / safe_w1, jnp.inf)
    order = jnp.argsort(theta)
    theta_s = theta[order]
    rank = jnp.argsort(order)
    k = jnp.arange(_NSEG)[:, None]
    active = jnp.where(w1[None, :] > 0, rank[None, :] < k,
                       jnp.where(w1[None, :] < 0, rank[None, :] >= k,
                                 b1[None, :] > 0))
    act = active.astype(f32)
    A = (act * w1[None, :]) @ W2c
    Bs = (act * b1[None, :]) @ W2c + btot
    Tws = (Tw[:, None, :] + Ts[None, :, :]).reshape(_WEATHER * _SEASON, _EDIM)
    J = (Tws[:, None, :] + Bs[None, :, :]).reshape(_NJ, _EDIM)
    return J, A, theta_s


def _pack_pairs(M):
    w16 = jax.lax.bitcast_convert_type(M.astype(jnp.bfloat16), jnp.uint16)
    w32 = w16.astype(jnp.uint32)
    packed = w32[:, :_NW2] | (w32[:, _NW2:] << 16)
    return jax.lax.bitcast_convert_type(packed, jnp.int32).reshape(-1)


def kernel(weather_ids, time_of_day, season_ids, W_weather, W_season,
           W_t1, b_t1, W_t2, b_t2, W_c, b_c):
    J, A, theta_s = _fold_tables(W_weather, W_season, W_t1, b_t1,
                                 W_t2, b_t2, W_c, b_c)
    wid = weather_ids.reshape(_P).astype(jnp.int32)
    sid = season_ids.reshape(_P).astype(jnp.int32)
    t = time_of_day.reshape(_P).astype(jnp.float32)
    jf = _pack_pairs(J)
    af = _pack_pairs(A)
    thb = jnp.broadcast_to(theta_s[:, None], (_TDIM, 16)).reshape(_TDIM * 16)
    out_w = _sc_call(wid, sid, t, jf, af, thb).reshape(_P // 4, 128)
    lo = jax.lax.bitcast_convert_type(jnp.left_shift(out_w, 16),
                                      jnp.float32).reshape(_P, _NW2)
    hi = jax.lax.bitcast_convert_type(out_w & jnp.int32(-65536),
                                      jnp.float32).reshape(_P, _NW2)
    return jnp.concatenate([lo, hi], axis=-1).reshape(_B, _L, _EDIM)
